# Initial kernel scaffold; baseline (speedup 1.0000x reference)
import jax
import jax.numpy as jnp
from jax.experimental import pallas as pl


def kernel(int_ids, species_table, move_table, ability_table, item_table):
    raise NotImplementedError("write your pallas kernel here")


# speedup vs baseline: 3.4445x; 3.4445x over previous
import functools
import jax
import jax.numpy as jnp
from jax import lax
from jax.experimental import pallas as pl
from jax.experimental.pallas import tpu as pltpu, tpu_sc as plsc

NC, NS, L = 2, 16, 16
NW = NC * NS
NSLOT = 4096 * 12
SPW = NSLOT // NW
CPS = 12
GR = 128
NSB = SPW // GR
NSTEP = NSB * CPS
NBUF = 4

OFF_M = 2048
OFF_A = OFF_M + 2000
OFF_I = OFF_A + 350
NROWS = OFF_I + 1000

_mesh = plsc.VectorSubcoreMesh(core_axis_name="c", subcore_axis_name="s")


@functools.partial(
    pl.kernel,
    out_type=jax.ShapeDtypeStruct((NSLOT, CPS, 64), jnp.float32),
    mesh=_mesh,
    scratch_types=[
        pltpu.VMEM((7, SPW), jnp.int32),
        pltpu.VMEM((CPS, SPW), jnp.int32),
        pltpu.VMEM((NBUF, GR, 64), jnp.float32),
        pltpu.SemaphoreType.DMA((NBUF,)),
    ],
    compiler_params=pltpu.CompilerParams(use_tc_tiling_on_sc=False),
)
def _embed(comb_hbm, ids_hbm, out_hbm, ids_v, idx_v, gbuf, sems):
    wid = lax.axis_index("s") * NC + lax.axis_index("c")
    pltpu.sync_copy(ids_hbm.at[wid], ids_v)

    @pl.loop(0, SPW // L)
    def _build(g):
        base = g * L
        i0 = ids_v[0, pl.ds(base, L)]
        i1 = ids_v[1, pl.ds(base, L)]
        i2 = ids_v[2, pl.ds(base, L)]
        i3 = ids_v[3, pl.ds(base, L)]
        i4 = ids_v[4, pl.ds(base, L)]
        i5 = ids_v[5, pl.ds(base, L)]
        i6 = ids_v[6, pl.ds(base, L)]
        vals = (
            i0 * 2, i0 * 2 + 1,
            OFF_M + i1 * 2, OFF_M + i1 * 2 + 1,
            OFF_M + i2 * 2, OFF_M + i2 * 2 + 1,
            OFF_M + i3 * 2, OFF_M + i3 * 2 + 1,
            OFF_M + i4 * 2, OFF_M + i4 * 2 + 1,
            OFF_A + i5,
            OFF_I + i6,
        )
        for ci, v in enumerate(vals):
            idx_v[ci, pl.ds(base, L)] = v

    slot0 = wid * SPW

    def _start(t, b):
        sb = t // CPS
        c = t % CPS
        pltpu.async_copy(
            comb_hbm.at[idx_v.at[c, pl.ds(sb * GR, GR)]],
            gbuf.at[b], sems.at[b])

    def _finish(t, b):
        sb = t // CPS
        c = t % CPS
        pltpu.make_async_copy(
            comb_hbm.at[idx_v.at[c, pl.ds(sb * GR, GR)]],
            gbuf.at[b], sems.at[b]).wait()
        pltpu.sync_copy(
            gbuf.at[b],
            out_hbm.at[pl.ds(slot0 + sb * GR, GR), c])

    for b in range(NBUF):
        _start(b, b)

    @pl.loop(0, NSTEP, step=NBUF)
    def _gather(t0):
        for b in range(NBUF):
            t = t0 + b
            _finish(t, b)
            nt = t + NBUF

            @pl.when(nt < NSTEP)
            def _():
                _start(nt, b)


def kernel(int_ids, species_table, move_table, ability_table, item_table):
    comb = jnp.concatenate([
        species_table.reshape(2048, 64),
        move_table.reshape(2000, 64),
        ability_table,
        item_table,
    ], axis=0)
    ids = int_ids.astype(jnp.int32).reshape(NW, SPW, 7).transpose(0, 2, 1)
    out = _embed(comb, ids)
    return out.reshape(4096, 12, 768)
